# Initial kernel scaffold; baseline (speedup 1.0000x reference)
import jax
import jax.numpy as jnp
from jax.experimental import pallas as pl


def kernel(x, edge_list, W1, b1, W2, b2, W3, b3, Wfc, bfc):
    raise NotImplementedError("write your pallas kernel here")


# speedup vs baseline: 9.5127x; 9.5127x over previous
import functools

import jax
import jax.numpy as jnp
from jax import lax
from jax.experimental import pallas as pl
from jax.experimental.pallas import tpu as pltpu
from jax.experimental.pallas import tpu_sc as plsc

N_PER_GRAPH = 1000
BATCH = 50
N = BATCH * N_PER_GRAPH
E = 800000
F_IN = 16
HID = 64

NC = 2
NS = 16
HALF = N // NC
CHUNK = 128
NCHUNK = E // CHUNK
INITROWS = 200
NINIT = HALF // INITROWS

_sc_mesh = plsc.VectorSubcoreMesh(core_axis_name="c", subcore_axis_name="s")
_sc_params = pltpu.CompilerParams(use_tc_tiling_on_sc=False)


def _edge_loop(sid, body):
    trips = (NCHUNK - 1) // NS + 1

    def step(t, carry):
        k = sid + t * NS

        @pl.when(k < NCHUNK)
        def _():
            body(k)

        return carry

    lax.fori_loop(0, trips, step, 0)


def _strided_loop(sid, n, body):
    trips = (n - 1) // NS + 1

    def step(t, carry):
        j = sid + t * NS

        @pl.when(j < n)
        def _():
            body(j)

        return carry

    lax.fori_loop(0, trips, step, 0)


DEGW = 16


@functools.partial(
    pl.kernel,
    out_type=jax.ShapeDtypeStruct((N, DEGW), jnp.float32),
    mesh=_sc_mesh,
    scratch_types=[
        pltpu.VMEM((CHUNK,), jnp.int32),
        pltpu.VMEM((CHUNK,), jnp.int32),
        pltpu.VMEM((CHUNK, DEGW), jnp.float32),
        pltpu.VMEM((INITROWS, DEGW), jnp.float32),
        pltpu.VMEM_SHARED((HALF + NS, DEGW), jnp.float32),
    ],
    compiler_params=_sc_params,
)
def _deg_sc(dst_hbm, deg_hbm, dst_v, dloc_v, ones_v, zeros_v, acc_sp):
    cid = lax.axis_index("c")
    sid = lax.axis_index("s")
    base = cid * HALF

    def fill(i, _):
        ones_v[i] = jnp.full((DEGW,), 1.0, jnp.float32)
        return _

    lax.fori_loop(0, CHUNK, fill, 0)

    def fillz(i, _):
        zeros_v[i] = jnp.zeros((DEGW,), jnp.float32)
        return _

    lax.fori_loop(0, INITROWS, fillz, 0)

    def zero_chunk(j):
        pltpu.sync_copy(zeros_v, acc_sp.at[pl.ds(j * INITROWS, INITROWS)])

    _strided_loop(sid, NINIT, zero_chunk)
    plsc.subcore_barrier()

    trash = HALF + sid

    def do_chunk(k):
        pltpu.sync_copy(dst_hbm.at[pl.ds(k * CHUNK, CHUNK)], dst_v)
        for j in range(CHUNK // 16):
            d = dst_v[pl.ds(j * 16, 16)]
            inr = (d >= base) & (d < base + HALF)
            dloc_v[pl.ds(j * 16, 16)] = jnp.where(inr, d - base, trash)
        pltpu.sync_copy(ones_v, acc_sp.at[dloc_v], add=True)

    _edge_loop(sid, do_chunk)
    plsc.subcore_barrier()

    def writeback(j):
        r = j * INITROWS
        pltpu.sync_copy(acc_sp.at[pl.ds(r, INITROWS)],
                        deg_hbm.at[pl.ds(base + r, INITROWS)])

    _strided_loop(sid, NINIT, writeback)


@functools.partial(
    pl.kernel,
    out_type=jax.ShapeDtypeStruct((N, HID), jnp.float32),
    mesh=_sc_mesh,
    scratch_types=[
        pltpu.VMEM((CHUNK,), jnp.int32),
        pltpu.VMEM((CHUNK,), jnp.int32),
        pltpu.VMEM((CHUNK,), jnp.int32),
        pltpu.VMEM((CHUNK, HID), jnp.float32),
        pltpu.VMEM_SHARED((HALF + NS, HID), jnp.float32),
        pltpu.SemaphoreType.DMA,
    ],
    compiler_params=_sc_params,
)
def _agg_sc(p_hbm, src_hbm, dst_hbm, q_hbm, src_v, dst_v, dloc_v, rows_v,
            q_sp, sem):
    cid = lax.axis_index("c")
    sid = lax.axis_index("s")
    base = cid * HALF

    def init_chunk(j):
        r = j * INITROWS
        pltpu.sync_copy(p_hbm.at[pl.ds(base + r, INITROWS)],
                        q_sp.at[pl.ds(r, INITROWS)])

    _strided_loop(sid, NINIT, init_chunk)
    plsc.subcore_barrier()

    trash = HALF + sid

    def do_chunk(k):
        e0 = k * CHUNK
        pltpu.sync_copy(src_hbm.at[pl.ds(e0, CHUNK)], src_v)
        pltpu.sync_copy(dst_hbm.at[pl.ds(e0, CHUNK)], dst_v)
        pltpu.async_copy(p_hbm.at[src_v], rows_v, sem).wait()
        for j in range(CHUNK // 16):
            d = dst_v[pl.ds(j * 16, 16)]
            inr = (d >= base) & (d < base + HALF)
            dloc_v[pl.ds(j * 16, 16)] = jnp.where(inr, d - base, trash)
        pltpu.sync_copy(rows_v, q_sp.at[dloc_v], add=True)

    _edge_loop(sid, do_chunk)
    plsc.subcore_barrier()

    def writeback(j):
        r = j * INITROWS
        pltpu.sync_copy(q_sp.at[pl.ds(r, INITROWS)],
                        q_hbm.at[pl.ds(base + r, INITROWS)])

    _strided_loop(sid, NINIT, writeback)


ROWS_TC = 5000


def _tc_first_body(x_ref, deg_ref, w_ref, p_ref):
    dinv = lax.rsqrt(deg_ref[:, 0:1] + 1.0)
    h = jnp.dot(x_ref[...], w_ref[...], preferred_element_type=jnp.float32)
    p_ref[...] = h * dinv


_tc_first = pl.pallas_call(
    _tc_first_body,
    grid=(N // ROWS_TC,),
    in_specs=[
        pl.BlockSpec((ROWS_TC, F_IN), lambda i: (i, 0)),
        pl.BlockSpec((ROWS_TC, DEGW), lambda i: (i, 0)),
        pl.BlockSpec((F_IN, HID), lambda i: (0, 0)),
    ],
    out_specs=pl.BlockSpec((ROWS_TC, HID), lambda i: (i, 0)),
    out_shape=jax.ShapeDtypeStruct((N, HID), jnp.float32),
)


def _tc_mid_body(q_ref, deg_ref, b_ref, w_ref, p_ref):
    dinv = lax.rsqrt(deg_ref[:, 0:1] + 1.0)
    t = q_ref[...] * dinv + b_ref[...]
    act = jnp.where(t > 0, t, 0.2 * t)
    h = jnp.dot(act, w_ref[...], preferred_element_type=jnp.float32)
    p_ref[...] = h * dinv


_tc_mid = pl.pallas_call(
    _tc_mid_body,
    grid=(N // ROWS_TC,),
    in_specs=[
        pl.BlockSpec((ROWS_TC, HID), lambda i: (i, 0)),
        pl.BlockSpec((ROWS_TC, DEGW), lambda i: (i, 0)),
        pl.BlockSpec((1, HID), lambda i: (0, 0)),
        pl.BlockSpec((HID, HID), lambda i: (0, 0)),
    ],
    out_specs=pl.BlockSpec((ROWS_TC, HID), lambda i: (i, 0)),
    out_shape=jax.ShapeDtypeStruct((N, HID), jnp.float32),
)


def _tc_final_body(q_ref, deg_ref, b_ref, wfc_ref, bfc_ref, out_ref):
    dinv = lax.rsqrt(deg_ref[:, 0:1] + 1.0)
    t = q_ref[...] * dinv + b_ref[...]
    act = jnp.where(t > 0, t, 0.2 * t)
    s = jnp.sum(act * wfc_ref[...]) + bfc_ref[0, 0]
    out_ref[...] = jnp.broadcast_to(jax.nn.sigmoid(s), (1, 1, 128))


_tc_final = pl.pallas_call(
    _tc_final_body,
    grid=(BATCH,),
    in_specs=[
        pl.BlockSpec((N_PER_GRAPH, HID), lambda g: (g, 0)),
        pl.BlockSpec((N_PER_GRAPH, DEGW), lambda g: (g, 0)),
        pl.BlockSpec((1, HID), lambda g: (0, 0)),
        pl.BlockSpec((N_PER_GRAPH, HID), lambda g: (0, 0)),
        pl.BlockSpec((1, 128), lambda g: (0, 0)),
    ],
    out_specs=pl.BlockSpec((1, 1, 128), lambda g: (g, 0, 0)),
    out_shape=jax.ShapeDtypeStruct((BATCH, 1, 128), jnp.float32),
)


def kernel(x, edge_list, W1, b1, W2, b2, W3, b3, Wfc, bfc):
    src = edge_list[0]
    dst = edge_list[1]
    b1r = b1.reshape(1, HID)
    b2r = b2.reshape(1, HID)
    b3r = b3.reshape(1, HID)
    wfcr = Wfc.reshape(N_PER_GRAPH, HID)
    bfcr = jnp.broadcast_to(bfc.reshape(1, 1), (1, 128))

    deg16 = _deg_sc(dst)
    p1 = _tc_first(x, deg16, W1)
    q1 = _agg_sc(p1, src, dst)
    p2 = _tc_mid(q1, deg16, b1r, W2)
    q2 = _agg_sc(p2, src, dst)
    p3 = _tc_mid(q2, deg16, b2r, W3)
    q3 = _agg_sc(p3, src, dst)
    out = _tc_final(q3, deg16, b3r, wfcr, bfcr)
    return out[:, 0, 0]


# speedup vs baseline: 9.5584x; 1.0048x over previous
import functools

import jax
import jax.numpy as jnp
from jax import lax
from jax.experimental import pallas as pl
from jax.experimental.pallas import tpu as pltpu
from jax.experimental.pallas import tpu_sc as plsc

N_PER_GRAPH = 1000
BATCH = 50
N = BATCH * N_PER_GRAPH
E = 800000
F_IN = 16
HID = 64
FH = HID // 2

NC = 2
NS = 16
CHUNK = 128
SUPC = 16
NSUP = 26
ROWS_PER_TILE = NSUP * SUPC
EROWS = ROWS_PER_TILE * NS
EPAD = EROWS * CHUNK
DEG_ROWS_PER_TILE = EROWS // (NC * NS)
DEG_NSUP = DEG_ROWS_PER_TILE // SUPC
INITROWS = 200
NINIT = N // INITROWS
DEGW = 16

_sc_mesh = plsc.VectorSubcoreMesh(core_axis_name="c", subcore_axis_name="s")
_sc_params = pltpu.CompilerParams(use_tc_tiling_on_sc=False)


def _strided_loop(sid, n, body):
    trips = (n - 1) // NS + 1

    def step(t, carry):
        j = sid + t * NS

        @pl.when(j < n)
        def _():
            body(j)

        return carry

    lax.fori_loop(0, trips, step, 0)


@functools.partial(
    pl.kernel,
    out_type=jax.ShapeDtypeStruct((NC * N, DEGW), jnp.float32),
    mesh=_sc_mesh,
    scratch_types=[
        pltpu.VMEM((SUPC, CHUNK), jnp.int32),
        pltpu.VMEM((CHUNK, DEGW), jnp.float32),
        pltpu.VMEM((INITROWS, DEGW), jnp.float32),
        pltpu.VMEM_SHARED((N + 8, DEGW), jnp.float32),
        pltpu.SemaphoreType.DMA,
    ],
    compiler_params=_sc_params,
)
def _deg_sc(dst_hbm, deg_hbm, dst_v, ones_v, zeros_v, acc_sp, sem):
    cid = lax.axis_index("c")
    sid = lax.axis_index("s")

    def fill(i, _):
        ones_v[i] = jnp.full((DEGW,), 1.0, jnp.float32)
        return _

    lax.fori_loop(0, CHUNK, fill, 0)

    def fillz(i, _):
        zeros_v[i] = jnp.zeros((DEGW,), jnp.float32)
        return _

    lax.fori_loop(0, INITROWS, fillz, 0)

    def zero_chunk(j):
        pltpu.sync_copy(zeros_v, acc_sp.at[pl.ds(j * INITROWS, INITROWS)])

    _strided_loop(sid, NINIT, zero_chunk)
    plsc.subcore_barrier()

    wrow = (cid * NS + sid) * DEG_ROWS_PER_TILE

    def do_super(s, carry):
        row0 = wrow + s * SUPC
        pltpu.sync_copy(dst_hbm.at[pl.ds(row0, SUPC)], dst_v)
        descs = []
        for c in range(SUPC):
            descs.append(
                pltpu.async_copy(ones_v, acc_sp.at[dst_v.at[c]], sem,
                                 add=True))
        for d in descs:
            d.wait()
        return carry

    lax.fori_loop(0, DEG_NSUP, do_super, 0)
    plsc.subcore_barrier()

    def writeback(j):
        r = j * INITROWS
        pltpu.sync_copy(acc_sp.at[pl.ds(r, INITROWS)],
                        deg_hbm.at[pl.ds(cid * N + r, INITROWS)])

    _strided_loop(sid, NINIT, writeback)


@functools.partial(
    pl.kernel,
    out_type=jax.ShapeDtypeStruct((NC * N, FH), jnp.float32),
    mesh=_sc_mesh,
    scratch_types=[
        pltpu.VMEM((2, SUPC, CHUNK), jnp.int32),
        pltpu.VMEM((2, SUPC, CHUNK), jnp.int32),
        pltpu.VMEM((2, CHUNK, FH), jnp.float32),
        pltpu.VMEM_SHARED((N + 8, FH), jnp.float32),
        pltpu.SemaphoreType.DMA,
        pltpu.SemaphoreType.DMA,
        pltpu.SemaphoreType.DMA,
        pltpu.SemaphoreType.DMA,
        pltpu.SemaphoreType.DMA,
        pltpu.SemaphoreType.DMA,
    ],
    compiler_params=_sc_params,
)
def _agg_sc(p_hbm, src_hbm, dst_hbm, q_hbm, src_v, dst_v, rows_v, acc_sp,
            ss0, ss1, sd0, sd1, sg0, sg1):
    cid = lax.axis_index("c")
    sid = lax.axis_index("s")
    ssem = (ss0, ss1)
    dsem = (sd0, sd1)
    gsem = (sg0, sg1)

    def init_chunk(j):
        r = j * INITROWS
        pltpu.sync_copy(p_hbm.at[pl.ds(cid * N + r, INITROWS)],
                        acc_sp.at[pl.ds(r, INITROWS)])

    _strided_loop(sid, NINIT, init_chunk)
    plsc.subcore_barrier()

    wrow = sid * ROWS_PER_TILE
    srow = cid * EROWS + wrow

    def issue_idx(s, slot):
        pltpu.async_copy(src_hbm.at[pl.ds(srow + s * SUPC, SUPC)],
                         src_v.at[slot], ssem[slot])
        pltpu.async_copy(dst_hbm.at[pl.ds(wrow + s * SUPC, SUPC)],
                         dst_v.at[slot], dsem[slot])

    issue_idx(0, 0)

    def do_pair(t, carry):
        for par in (0, 1):
            s = 2 * t + par
            pltpu.make_async_copy(src_hbm.at[pl.ds(0, SUPC)], src_v.at[par],
                                  ssem[par]).wait()
            pltpu.make_async_copy(dst_hbm.at[pl.ds(0, SUPC)], dst_v.at[par],
                                  dsem[par]).wait()

            @pl.when(s + 1 < NSUP)
            def _():
                issue_idx(s + 1, 1 - par)

            descs = [None, None]
            descs[0] = pltpu.async_copy(p_hbm.at[src_v.at[par, 0]],
                                        rows_v.at[0], gsem[0])
            for c in range(1, SUPC):
                rb = c & 1
                descs[rb] = pltpu.async_copy(p_hbm.at[src_v.at[par, c]],
                                             rows_v.at[rb], gsem[rb])
                descs[1 - rb].wait()
                pltpu.sync_copy(rows_v.at[1 - rb],
                                acc_sp.at[dst_v.at[par, c - 1]], add=True)
            descs[1].wait()
            pltpu.sync_copy(rows_v.at[1],
                            acc_sp.at[dst_v.at[par, SUPC - 1]], add=True)
        return carry

    lax.fori_loop(0, NSUP // 2, do_pair, 0)
    plsc.subcore_barrier()

    def writeback(j):
        r = j * INITROWS
        pltpu.sync_copy(acc_sp.at[pl.ds(r, INITROWS)],
                        q_hbm.at[pl.ds(cid * N + r, INITROWS)])

    _strided_loop(sid, NINIT, writeback)


ROWS_TC = 5000
NB = N // ROWS_TC


def _dinv_of(dega_ref, degb_ref):
    return lax.rsqrt(dega_ref[:, 0:1] + degb_ref[:, 0:1] + 1.0)


def _leaky(t):
    return jnp.where(t > 0, t, 0.2 * t)


def _half_sel(h, c):
    return jnp.where(c == 0, h[:, :FH], h[:, FH:])


def _tc_first_body(x_ref, dega_ref, degb_ref, w_ref, p_ref):
    c = pl.program_id(0)
    dinv = _dinv_of(dega_ref, degb_ref)
    h = jnp.dot(x_ref[...], w_ref[...], preferred_element_type=jnp.float32)
    p_ref[...] = _half_sel(h, c) * dinv


_tc_first = pl.pallas_call(
    _tc_first_body,
    grid=(NC, NB),
    in_specs=[
        pl.BlockSpec((ROWS_TC, F_IN), lambda c, i: (i, 0)),
        pl.BlockSpec((ROWS_TC, DEGW), lambda c, i: (i, 0)),
        pl.BlockSpec((ROWS_TC, DEGW), lambda c, i: (i + NB, 0)),
        pl.BlockSpec((F_IN, HID), lambda c, i: (0, 0)),
    ],
    out_specs=pl.BlockSpec((ROWS_TC, FH), lambda c, i: (c * NB + i, 0)),
    out_shape=jax.ShapeDtypeStruct((NC * N, FH), jnp.float32),
)


def _tc_mid_body(qa_ref, qb_ref, dega_ref, degb_ref, b_ref, w_ref, p_ref):
    c = pl.program_id(0)
    dinv = _dinv_of(dega_ref, degb_ref)
    act0 = _leaky(qa_ref[...] * dinv + b_ref[0:1, :])
    act1 = _leaky(qb_ref[...] * dinv + b_ref[1:2, :])
    act = jnp.concatenate([act0, act1], axis=1)
    h = jnp.dot(act, w_ref[...], preferred_element_type=jnp.float32)
    p_ref[...] = _half_sel(h, c) * dinv


_tc_mid = pl.pallas_call(
    _tc_mid_body,
    grid=(NC, NB),
    in_specs=[
        pl.BlockSpec((ROWS_TC, FH), lambda c, i: (i, 0)),
        pl.BlockSpec((ROWS_TC, FH), lambda c, i: (i + NB, 0)),
        pl.BlockSpec((ROWS_TC, DEGW), lambda c, i: (i, 0)),
        pl.BlockSpec((ROWS_TC, DEGW), lambda c, i: (i + NB, 0)),
        pl.BlockSpec((NC, FH), lambda c, i: (0, 0)),
        pl.BlockSpec((HID, HID), lambda c, i: (0, 0)),
    ],
    out_specs=pl.BlockSpec((ROWS_TC, FH), lambda c, i: (c * NB + i, 0)),
    out_shape=jax.ShapeDtypeStruct((NC * N, FH), jnp.float32),
)


def _tc_final_body(qa_ref, qb_ref, dega_ref, degb_ref, b_ref, wfca_ref,
                   wfcb_ref, bfc_ref, out_ref):
    dinv = _dinv_of(dega_ref, degb_ref)
    act0 = _leaky(qa_ref[...] * dinv + b_ref[0:1, :])
    act1 = _leaky(qb_ref[...] * dinv + b_ref[1:2, :])
    s = (jnp.sum(act0 * wfca_ref[...]) + jnp.sum(act1 * wfcb_ref[...])
         + bfc_ref[0, 0])
    out_ref[...] = jnp.broadcast_to(jax.nn.sigmoid(s), (1, 1, 128))


GB = N_PER_GRAPH
NGB = N // GB

_tc_final = pl.pallas_call(
    _tc_final_body,
    grid=(BATCH,),
    in_specs=[
        pl.BlockSpec((GB, FH), lambda g: (g, 0)),
        pl.BlockSpec((GB, FH), lambda g: (g + NGB, 0)),
        pl.BlockSpec((GB, DEGW), lambda g: (g, 0)),
        pl.BlockSpec((GB, DEGW), lambda g: (g + NGB, 0)),
        pl.BlockSpec((NC, FH), lambda g: (0, 0)),
        pl.BlockSpec((GB, FH), lambda g: (0, 0)),
        pl.BlockSpec((GB, FH), lambda g: (1, 0)),
        pl.BlockSpec((1, 128), lambda g: (0, 0)),
    ],
    out_specs=pl.BlockSpec((1, 1, 128), lambda g: (g, 0, 0)),
    out_shape=jax.ShapeDtypeStruct((BATCH, 1, 128), jnp.float32),
)


def kernel(x, edge_list, W1, b1, W2, b2, W3, b3, Wfc, bfc):
    npad = EPAD - E
    src2d = jnp.concatenate(
        [edge_list[0], jnp.zeros((npad,), jnp.int32)]).reshape(EROWS, CHUNK)
    dst2d = jnp.concatenate(
        [edge_list[1], jnp.full((npad,), N, jnp.int32)]).reshape(EROWS, CHUNK)
    src_both = jnp.concatenate([src2d, src2d + N], axis=0)
    b1s = b1.reshape(NC, FH)
    b2s = b2.reshape(NC, FH)
    b3s = b3.reshape(NC, FH)
    wfc2 = Wfc.reshape(GB, NC, FH).transpose(1, 0, 2).reshape(NC * GB, FH)
    bfcr = jnp.broadcast_to(bfc.reshape(1, 1), (1, 128))

    deg2 = _deg_sc(dst2d)
    p1 = _tc_first(x, deg2, deg2, W1)
    q1 = _agg_sc(p1, src_both, dst2d)
    p2 = _tc_mid(q1, q1, deg2, deg2, b1s, W2)
    q2 = _agg_sc(p2, src_both, dst2d)
    p3 = _tc_mid(q2, q2, deg2, deg2, b2s, W3)
    q3 = _agg_sc(p3, src_both, dst2d)
    out = _tc_final(q3, q3, deg2, deg2, b3s, wfc2, wfc2, bfcr)
    return out[:, 0, 0]
